# baseline (speedup 1.0000x reference)
import jax
import jax.numpy as jnp
from jax import lax
from jax.experimental import pallas as pl
from jax.experimental.pallas import tpu as pltpu
from jax.experimental.pallas import tpu_sc as plsc

VOCAB = 1000
BATCH = 1024
SEQ = 50
NC, NS = 2, 16
NW = NC * NS
NPIECE = 5
PCOLS = VOCAB // NPIECE
PVT = PCOLS // 8
NUNITS = SEQ * 8
KMAX = 13
MAXP = KMAX * NPIECE
OUTW = (VOCAB // 8) * 8192


def _body(table5_hbm, idxu_hbm, out_hbm, units_v, col5, gidx, rows, stage,
          usem, gsem, ssem):
    w = lax.axis_index("s") * NC + lax.axis_index("c")
    iota = lax.iota(jnp.int32, 16)
    zeros = iota * 0
    npw = jnp.where(w < NUNITS - KMAX * NW + NW, MAXP, MAXP - NPIECE)

    uids = jnp.minimum(w + NW * iota, NUNITS - 1)
    units_v[pl.ds(0, 16)] = uids
    pltpu.async_copy(idxu_hbm.at[units_v], col5, usem).wait()
    for k in range(KMAX):
        for g in range(8):
            col5[k, pl.ds(16 * g, 16)] = col5[k, pl.ds(16 * g, 16)] * 5

    diag = [(iota + r) & 15 for r in range(16)]
    dhi = [d >> 3 for d in diag]
    dlo = [(d & 7) * 128 + iota for d in diag]
    diag_t = [jnp.minimum(192 + d, PCOLS - 1) for d in diag]
    msk_t = [d < 8 for d in diag]

    def prep_and_fire(P, b):
        k = P // NPIECE
        p = P % NPIECE
        for g in range(8):
            gidx[b][pl.ds(16 * g, 16)] = col5[k, pl.ds(16 * g, 16)] + p
        pltpu.async_copy(table5_hbm.at[gidx[b]], rows[b], gsem[b])

    def wait_gather(b):
        pltpu.make_async_copy(
            table5_hbm.at[gidx[b]], rows[b], gsem[b]
        ).wait()

    def fire_stores(P, b):
        k = P // NPIECE
        p = P % NPIECE
        u = w + NW * k
        t = u // 8
        bt = u % 8
        pltpu.async_copy(
            stage[b],
            out_hbm.at[t, pl.ds(PVT * p, PVT), pl.ds(1024 * bt, 1024)],
            ssem[b],
        )

    def drain_stores(b):
        pltpu.make_async_copy(
            stage[b],
            out_hbm.at[0, pl.ds(0, PVT), pl.ds(0, 1024)],
            ssem[b],
        ).wait()

    def transpose(b):
        @plsc.parallel_loop(0, 128, step=16)
        def _(bl0):
            rowv = iota + bl0

            @plsc.parallel_loop(0, 192, step=16, unroll=2)
            def _(c0):
                vt0 = c0 >> 3
                for r in range(16):
                    x = plsc.load_gather(rows[b], [rowv, diag[r] + c0])
                    plsc.store_scatter(
                        stage[b], [dhi[r] + vt0, dlo[r] + bl0], x
                    )

            for r in range(16):
                x = plsc.load_gather(rows[b], [rowv, diag_t[r]],
                                     mask=msk_t[r])
                plsc.store_scatter(
                    stage[b], [zeros + 24, dlo[r] + bl0], x, mask=msk_t[r]
                )

    prep_and_fire(0, 0)

    @pl.loop(0, MAXP + 1, step=2)
    def _(P0):
        for d in range(2):
            P = P0 + d
            b = d

            @pl.when(P < npw)
            def _():
                wait_gather(b)

                @pl.when(P + 1 < npw)
                def _():
                    prep_and_fire(P + 1, 1 - b)

                @pl.when(P >= 2)
                def _():
                    drain_stores(b)

                transpose(b)
                fire_stores(P, b)

    drain_stores(0)
    drain_stores(1)


@jax.jit
def _lookup(idx, table):
    idxu = idx.T.reshape(SEQ * 8, 128).astype(jnp.int32)
    table5 = table.reshape(VOCAB * NPIECE, PCOLS)
    mesh = plsc.VectorSubcoreMesh(core_axis_name="c", subcore_axis_name="s")
    run = pl.kernel(
        _body,
        out_type=jax.ShapeDtypeStruct((SEQ, VOCAB // 8, 8192), jnp.float32),
        mesh=mesh,
        compiler_params=pltpu.CompilerParams(
            use_tc_tiling_on_sc=False, needs_layout_passes=False
        ),
        scratch_types=[
            pltpu.VMEM((16,), jnp.int32),
            pltpu.VMEM((16, 128), jnp.int32),
            [pltpu.VMEM((128,), jnp.int32) for _ in range(2)],
            [pltpu.VMEM((128, PCOLS), jnp.float32) for _ in range(2)],
            [pltpu.VMEM((PVT, 1024), jnp.float32) for _ in range(2)],
            pltpu.SemaphoreType.DMA,
            [pltpu.SemaphoreType.DMA for _ in range(2)],
            [pltpu.SemaphoreType.DMA for _ in range(2)],
        ],
    )
    L = run(table5, idxu)
    L5 = L.reshape(SEQ, VOCAB // 8, 8, 8, 128)
    return L5.transpose(2, 4, 0, 1, 3).reshape(BATCH, SEQ, VOCAB)


def kernel(idx, table):
    return _lookup(idx, table)


# speedup vs baseline: 1.4765x; 1.4765x over previous
import jax
import jax.numpy as jnp
from jax import lax
from jax.experimental import pallas as pl
from jax.experimental.pallas import tpu as pltpu
from jax.experimental.pallas import tpu_sc as plsc

VOCAB = 1000
BATCH = 1024
SEQ = 50
NC, NS = 2, 16
NW = NC * NS
NPIECE = 5
PCOLS = VOCAB // NPIECE
PVT = PCOLS // 8
NUNITS = SEQ * 8
KMAX = 13
MAXP = KMAX * NPIECE
OUTW = (VOCAB // 8) * 8192


def _body(table5_hbm, idxu_hbm, out_hbm, units_v, col5, gidx, rows, stage,
          usem, gsem, ssem):
    w = lax.axis_index("s") * NC + lax.axis_index("c")
    iota = lax.iota(jnp.int32, 16)
    zeros = iota * 0
    npw = jnp.where(w < NUNITS - KMAX * NW + NW, MAXP, MAXP - NPIECE)

    uids = jnp.minimum(w + NW * iota, NUNITS - 1)
    units_v[pl.ds(0, 16)] = uids
    pltpu.async_copy(idxu_hbm.at[units_v], col5, usem).wait()
    for k in range(KMAX):
        for g in range(8):
            col5[k, pl.ds(16 * g, 16)] = col5[k, pl.ds(16 * g, 16)] * 5

    diag = [(iota + r) & 15 for r in range(16)]
    qdiag = [(d >> 3) * 1024 + (d & 7) * 128 + iota for d in diag]
    diag_t = [jnp.minimum(192 + d, PCOLS - 1) for d in diag]
    msk_t = [d < 8 for d in diag]

    def prep_and_fire(P, b):
        k = P // NPIECE
        p = P % NPIECE
        for g in range(8):
            gidx[b][pl.ds(16 * g, 16)] = col5[k, pl.ds(16 * g, 16)] + p
        pltpu.async_copy(table5_hbm.at[gidx[b]], rows[b], gsem[b])

    def wait_gather(b):
        pltpu.make_async_copy(
            table5_hbm.at[gidx[b]], rows[b], gsem[b]
        ).wait()

    def fire_stores(P, b):
        k = P // NPIECE
        p = P % NPIECE
        u = w + NW * k
        t = u // 8
        bt = u % 8
        base = (PVT * p) * 8192 + 1024 * bt
        for vtl in range(PVT):
            pltpu.async_copy(
                stage[b].at[pl.ds(1024 * vtl, 1024)],
                out_hbm.at[t, pl.ds(base + 8192 * vtl, 1024)],
                ssem[b],
            )

    def drain_stores(b):
        for vtl in range(PVT):
            pltpu.make_async_copy(
                stage[b].at[pl.ds(1024 * vtl, 1024)],
                out_hbm.at[0, pl.ds(8192 * vtl, 1024)],
                ssem[b],
            ).wait()

    def transpose(b):
        @plsc.parallel_loop(0, 128, step=16)
        def _(bl0):
            rowv = iota + bl0

            @plsc.parallel_loop(0, 192, step=16, unroll=2)
            def _(c0):
                s = c0 * 128 + bl0
                for r in range(16):
                    x = plsc.load_gather(rows[b], [rowv, diag[r] + c0])
                    plsc.store_scatter(stage[b], [qdiag[r] + s], x)

            s = 192 * 128 + bl0
            for r in range(16):
                x = plsc.load_gather(rows[b], [rowv, diag_t[r]],
                                     mask=msk_t[r])
                plsc.store_scatter(stage[b], [qdiag[r] + s], x,
                                   mask=msk_t[r])

    prep_and_fire(0, 0)

    @pl.loop(0, MAXP + 1, step=2)
    def _(P0):
        for d in range(2):
            P = P0 + d
            b = d

            @pl.when(P < npw)
            def _():
                wait_gather(b)

                @pl.when(P + 1 < npw)
                def _():
                    prep_and_fire(P + 1, 1 - b)

                @pl.when(P >= 2)
                def _():
                    drain_stores(b)

                transpose(b)
                fire_stores(P, b)

    drain_stores(0)
    drain_stores(1)


@jax.jit
def _lookup(idx, table):
    idxu = idx.T.reshape(SEQ * 8, 128).astype(jnp.int32)
    table5 = table.reshape(VOCAB * NPIECE, PCOLS)
    mesh = plsc.VectorSubcoreMesh(core_axis_name="c", subcore_axis_name="s")
    run = pl.kernel(
        _body,
        out_type=jax.ShapeDtypeStruct((SEQ, OUTW), jnp.float32),
        mesh=mesh,
        compiler_params=pltpu.CompilerParams(
            use_tc_tiling_on_sc=False, needs_layout_passes=False
        ),
        scratch_types=[
            pltpu.VMEM((16,), jnp.int32),
            pltpu.VMEM((16, 128), jnp.int32),
            [pltpu.VMEM((128,), jnp.int32) for _ in range(2)],
            [pltpu.VMEM((128, PCOLS), jnp.float32) for _ in range(2)],
            [pltpu.VMEM((PVT * 1024,), jnp.float32) for _ in range(2)],
            pltpu.SemaphoreType.DMA,
            [pltpu.SemaphoreType.DMA for _ in range(2)],
            [pltpu.SemaphoreType.DMA for _ in range(2)],
        ],
    )
    L = run(table5, idxu)
    L5 = L.reshape(SEQ, VOCAB // 8, 8, 8, 128)
    return L5.transpose(2, 4, 0, 1, 3).reshape(BATCH, SEQ, VOCAB)


def kernel(idx, table):
    return _lookup(idx, table)
